# baseline (speedup 1.0000x reference)
import functools

import jax
import jax.numpy as jnp
from jax import lax
from jax.experimental import pallas as pl
from jax.experimental.pallas import tpu as pltpu
from jax.experimental.pallas import tpu_sc as plsc

NC = 2
NS = 16
L = 16

K = 64
GCH = 8
NG = 2
NSB = 2


def _io_tiles(n):
    for t in range(NS, 0, -1):
        if n % t == 0 and (n // t) % 8 == 0:
            return t
    raise ValueError(n)


def _make_sc_spmm(n, d, e_pad):
    NW = NC * NS
    assert e_pad % (NW * K) == 0
    nch = e_pad // (NW * K)
    assert nch % GCH == 0
    ngr = nch // GCH
    nio = _io_tiles(n)
    rows_per_io = n // nio
    mesh = plsc.VectorSubcoreMesh(core_axis_name="c", subcore_axis_name="s")
    zrows = n // NS

    @functools.partial(
        pl.kernel,
        out_type=jax.ShapeDtypeStruct((NC, n, d), jnp.float32),
        mesh=mesh,
        scratch_types=[
            pltpu.VMEM((GCH, K), jnp.int32),
            pltpu.VMEM((GCH, K), jnp.int32),
            pltpu.VMEM((GCH * K // 8, L), jnp.float32),
            [pltpu.VMEM((K, d), jnp.float32)] * NG,
            [pltpu.VMEM((K, d), jnp.float32)] * NSB,
            pltpu.VMEM_SHARED((n, d), jnp.float32),
            [pltpu.SemaphoreType.DMA] * NG,
            [pltpu.SemaphoreType.DMA] * NSB,
        ],
    )
    def spmm(x_hbm, cols_hbm, rows_hbm, w_hbm, out_hbm,
             cols_v, rows_v, w_v, G, S, acc, GSEM, SSEM):
        cid = lax.axis_index("c")
        sid = lax.axis_index("s")
        tile = cid * NS + sid
        xh = x_hbm

        zero16 = jnp.zeros((L,), jnp.float32)

        def zero_row(r, c):
            for v in range(d // L):
                S[0][r, pl.ds(v * L, L)] = zero16
            return c

        lax.fori_loop(0, K, zero_row, 0, unroll=False)
        r0 = sid * zrows
        nfull = zrows // K
        for z in range(nfull):
            pltpu.sync_copy(S[0], acc.at[pl.ds(r0 + z * K, K), :])
        rem = zrows - nfull * K
        if rem:
            pltpu.sync_copy(S[0].at[pl.ds(0, rem), :],
                            acc.at[pl.ds(r0 + nfull * K, rem), :])
        plsc.subcore_barrier()

        def scale(jl, b, c):
            def group_body(g8, c2):
                w16 = w_v[jl * (K // 8) + g8, :]
                for e8 in range(8):
                    ei = g8 * 8 + e8
                    w = w16[e8]
                    for v in range(d // L):
                        sl = pl.ds(v * L, L)
                        S[c][ei, sl] = G[b][ei, sl] * w
                return c2

            lax.fori_loop(0, K // 8, group_body, 0, unroll=False)

        def group(g, carry_g):
            base = pl.multiple_of(tile * nch + g * GCH, 8)
            pltpu.sync_copy(cols_hbm.at[pl.ds(base, GCH)], cols_v)
            pltpu.sync_copy(rows_hbm.at[pl.ds(base, GCH)], rows_v)
            pltpu.sync_copy(
                w_hbm.at[pl.ds(pl.multiple_of(base * (K // 8), 8),
                               GCH * (K // 8))], w_v)
            for b in range(NG):
                pltpu.async_copy(xh.at[cols_v.at[b]], G[b], GSEM[b])

            def ring_body(i, carry):
                for b in range(NG):
                    jl = i * NG + b
                    c = b % NSB
                    pltpu.make_async_copy(
                        xh.at[cols_v.at[jl]], G[b], GSEM[b]).wait()

                    def _wait_scatter():
                        pltpu.make_async_copy(
                            S[c], acc.at[rows_v.at[jl]], SSEM[c]).wait()
                    if b < NSB:
                        pl.when(i > 0)(_wait_scatter)
                    else:
                        _wait_scatter()

                    scale(jl, b, c)

                    @pl.when(i < GCH // NG - 1)
                    def _prefetch():
                        pltpu.async_copy(
                            xh.at[cols_v.at[jl + NG]], G[b], GSEM[b])

                    pltpu.async_copy(S[c], acc.at[rows_v.at[jl]], SSEM[c],
                                     add=True)
                return carry

            lax.fori_loop(0, GCH // NG, ring_body, 0, unroll=False)

            for c in range(NSB):
                pltpu.make_async_copy(
                    S[c], acc.at[rows_v.at[GCH - NSB + c]], SSEM[c]).wait()
            return carry_g

        lax.fori_loop(0, ngr, group, 0, unroll=False)
        plsc.subcore_barrier()

        @pl.when(sid < nio)
        def _export():
            e0 = sid * rows_per_io
            pltpu.sync_copy(acc.at[pl.ds(e0, rows_per_io), :],
                            out_hbm.at[cid, pl.ds(e0, rows_per_io), :])

    return spmm


def _combine_body(p_ref, f_ref, l1f_ref, inter_ref):
    lf = p_ref[0] + p_ref[1]
    f = f_ref[...]
    l1f_ref[...] = lf + f
    inter_ref[...] = lf * f


def _final_body(l1f_ref, q_ref, w1_ref, w2_ref, b_ref, o_ref):
    li = q_ref[0] + q_ref[1]
    o_ref[...] = (
        jnp.dot(l1f_ref[...], w1_ref[...], preferred_element_type=jnp.float32)
        + jnp.dot(li, w2_ref[...], preferred_element_type=jnp.float32)
        + b_ref[...]
    )


def _tc_combine(parts, feats, bn):
    n, d = feats.shape
    grid = (n // bn,)
    return pl.pallas_call(
        _combine_body,
        grid=grid,
        in_specs=[
            pl.BlockSpec((2, bn, d), lambda i: (0, i, 0)),
            pl.BlockSpec((bn, d), lambda i: (i, 0)),
        ],
        out_specs=[
            pl.BlockSpec((bn, d), lambda i: (i, 0)),
            pl.BlockSpec((bn, d), lambda i: (i, 0)),
        ],
        out_shape=[
            jax.ShapeDtypeStruct((n, d), jnp.float32),
            jax.ShapeDtypeStruct((n, d), jnp.float32),
        ],
    )(parts, feats)


def _tc_final(l1f, q, W1, W2, b, bn):
    n, d = l1f.shape
    grid = (n // bn,)
    return pl.pallas_call(
        _final_body,
        grid=grid,
        in_specs=[
            pl.BlockSpec((bn, d), lambda i: (i, 0)),
            pl.BlockSpec((2, bn, d), lambda i: (0, i, 0)),
            pl.BlockSpec((d, d), lambda i: (0, 0)),
            pl.BlockSpec((d, d), lambda i: (0, 0)),
            pl.BlockSpec((1, d), lambda i: (0, 0)),
        ],
        out_specs=pl.BlockSpec((bn, d), lambda i: (i, 0)),
        out_shape=jax.ShapeDtypeStruct((n, d), jnp.float32),
    )(l1f, q, W1, W2, b)


@jax.jit
def kernel(edge_index, edge_weight, features, W1, b1, W2, b2):
    n, d = features.shape
    e = edge_index.shape[1]

    nch = -(-e // (NC * NS * K * GCH)) * GCH
    e_pad = NC * NS * K * nch
    pe = e_pad - e
    rows = jnp.concatenate([edge_index[0], jnp.zeros((pe,), edge_index.dtype)])
    cols = jnp.concatenate([edge_index[1], jnp.zeros((pe,), edge_index.dtype)])
    w = jnp.concatenate([edge_weight, jnp.zeros((pe,), edge_weight.dtype)])
    rows2d = rows.reshape(e_pad // K, K)
    cols2d = cols.reshape(e_pad // K, K)
    w8 = w.reshape(e_pad // 8, 8)
    w2d = jnp.concatenate([w8, w8], axis=1)

    spmm = _make_sc_spmm(n, d, e_pad)
    parts1 = spmm(features, cols2d, rows2d, w2d)

    bn = 1000 if n % 1000 == 0 else n
    l1f, inter = _tc_combine(parts1, features, bn)

    parts2 = spmm(inter, cols2d, rows2d, w2d)

    b = (b1 + b2).reshape(1, d)
    return _tc_final(l1f, parts2, W1, W2, b, bn)


# speedup vs baseline: 1.0586x; 1.0586x over previous
import functools

import jax
import jax.numpy as jnp
from jax import lax
from jax.experimental import pallas as pl
from jax.experimental.pallas import tpu as pltpu
from jax.experimental.pallas import tpu_sc as plsc

NC = 2
NS = 16
L = 16

K = 32
GCH = 32
NG = 2
NSB = 2


def _io_tiles(n):
    for t in range(NS, 0, -1):
        if n % t == 0 and (n // t) % 8 == 0:
            return t
    raise ValueError(n)


def _make_sc_spmm(n, d, e_pad):
    NW = NC * NS
    assert e_pad % (NW * K) == 0
    nch = e_pad // (NW * K)
    assert nch % GCH == 0
    ngr = nch // GCH
    nio = _io_tiles(n)
    rows_per_io = n // nio
    mesh = plsc.VectorSubcoreMesh(core_axis_name="c", subcore_axis_name="s")
    zrows = n // NS

    @functools.partial(
        pl.kernel,
        out_type=jax.ShapeDtypeStruct((NC, n, d), jnp.float32),
        mesh=mesh,
        scratch_types=[
            pltpu.VMEM((GCH, K), jnp.int32),
            pltpu.VMEM((GCH, K), jnp.int32),
            pltpu.VMEM((GCH * K // 8, L), jnp.float32),
            [pltpu.VMEM((K, d), jnp.float32)] * NG,
            [pltpu.VMEM((K, d), jnp.float32)] * NSB,
            pltpu.VMEM_SHARED((n, d), jnp.float32),
            [pltpu.SemaphoreType.DMA] * NG,
            [pltpu.SemaphoreType.DMA] * NSB,
        ],
    )
    def spmm(x_hbm, cols_hbm, rows_hbm, w_hbm, out_hbm,
             cols_v, rows_v, w_v, G, S, acc, GSEM, SSEM):
        cid = lax.axis_index("c")
        sid = lax.axis_index("s")
        tile = cid * NS + sid
        xh = x_hbm

        zero16 = jnp.zeros((L,), jnp.float32)

        def zero_row(r, c):
            for v in range(d // L):
                S[0][r, pl.ds(v * L, L)] = zero16
            return c

        lax.fori_loop(0, K, zero_row, 0, unroll=False)
        r0 = sid * zrows
        nfull = zrows // K
        for z in range(nfull):
            pltpu.sync_copy(S[0], acc.at[pl.ds(r0 + z * K, K), :])
        rem = zrows - nfull * K
        if rem:
            pltpu.sync_copy(S[0].at[pl.ds(0, rem), :],
                            acc.at[pl.ds(r0 + nfull * K, rem), :])
        plsc.subcore_barrier()

        def scale(jl, b, c):
            def group_body(g8, c2):
                w16 = w_v[jl * (K // 8) + g8, :]
                for e8 in range(8):
                    ei = g8 * 8 + e8
                    w = w16[e8]
                    for v in range(d // L):
                        sl = pl.ds(v * L, L)
                        S[c][ei, sl] = G[b][ei, sl] * w
                return c2

            lax.fori_loop(0, K // 8, group_body, 0, unroll=False)

        def group(g, carry_g):
            base = pl.multiple_of(tile * nch + g * GCH, 8)
            pltpu.sync_copy(cols_hbm.at[pl.ds(base, GCH)], cols_v)
            pltpu.sync_copy(rows_hbm.at[pl.ds(base, GCH)], rows_v)
            pltpu.sync_copy(
                w_hbm.at[pl.ds(pl.multiple_of(base * (K // 8), 8),
                               GCH * (K // 8))], w_v)
            for b in range(NG):
                pltpu.async_copy(xh.at[cols_v.at[b]], G[b], GSEM[b])

            def ring_body(i, carry):
                for b in range(NG):
                    jl = i * NG + b
                    c = b % NSB
                    pltpu.make_async_copy(
                        xh.at[cols_v.at[jl]], G[b], GSEM[b]).wait()

                    def _wait_scatter():
                        pltpu.make_async_copy(
                            S[c], acc.at[rows_v.at[jl]], SSEM[c]).wait()
                    if b < NSB:
                        pl.when(i > 0)(_wait_scatter)
                    else:
                        _wait_scatter()

                    scale(jl, b, c)

                    @pl.when(i < GCH // NG - 1)
                    def _prefetch():
                        pltpu.async_copy(
                            xh.at[cols_v.at[jl + NG]], G[b], GSEM[b])

                    pltpu.async_copy(S[c], acc.at[rows_v.at[jl]], SSEM[c],
                                     add=True)
                return carry

            lax.fori_loop(0, GCH // NG, ring_body, 0, unroll=False)

            for c in range(NSB):
                pltpu.make_async_copy(
                    S[c], acc.at[rows_v.at[GCH - NSB + c]], SSEM[c]).wait()
            return carry_g

        lax.fori_loop(0, ngr, group, 0, unroll=False)
        plsc.subcore_barrier()

        @pl.when(sid < nio)
        def _export():
            e0 = sid * rows_per_io
            pltpu.sync_copy(acc.at[pl.ds(e0, rows_per_io), :],
                            out_hbm.at[cid, pl.ds(e0, rows_per_io), :])

    return spmm


def _combine_body(p_ref, f_ref, l1f_ref, inter_ref):
    lf = p_ref[0] + p_ref[1]
    f = f_ref[...]
    l1f_ref[...] = lf + f
    inter_ref[...] = lf * f


def _final_body(l1f_ref, q_ref, w1_ref, w2_ref, b_ref, o_ref):
    li = q_ref[0] + q_ref[1]
    o_ref[...] = (
        jnp.dot(l1f_ref[...], w1_ref[...], preferred_element_type=jnp.float32)
        + jnp.dot(li, w2_ref[...], preferred_element_type=jnp.float32)
        + b_ref[...]
    )


def _tc_combine(parts, feats, bn):
    n, d = feats.shape
    grid = (n // bn,)
    return pl.pallas_call(
        _combine_body,
        grid=grid,
        in_specs=[
            pl.BlockSpec((2, bn, d), lambda i: (0, i, 0)),
            pl.BlockSpec((bn, d), lambda i: (i, 0)),
        ],
        out_specs=[
            pl.BlockSpec((bn, d), lambda i: (i, 0)),
            pl.BlockSpec((bn, d), lambda i: (i, 0)),
        ],
        out_shape=[
            jax.ShapeDtypeStruct((n, d), jnp.float32),
            jax.ShapeDtypeStruct((n, d), jnp.float32),
        ],
    )(parts, feats)


def _tc_final(l1f, q, W1, W2, b, bn):
    n, d = l1f.shape
    grid = (n // bn,)
    return pl.pallas_call(
        _final_body,
        grid=grid,
        in_specs=[
            pl.BlockSpec((bn, d), lambda i: (i, 0)),
            pl.BlockSpec((2, bn, d), lambda i: (0, i, 0)),
            pl.BlockSpec((d, d), lambda i: (0, 0)),
            pl.BlockSpec((d, d), lambda i: (0, 0)),
            pl.BlockSpec((1, d), lambda i: (0, 0)),
        ],
        out_specs=pl.BlockSpec((bn, d), lambda i: (i, 0)),
        out_shape=jax.ShapeDtypeStruct((n, d), jnp.float32),
    )(l1f, q, W1, W2, b)


@jax.jit
def kernel(edge_index, edge_weight, features, W1, b1, W2, b2):
    n, d = features.shape
    e = edge_index.shape[1]

    nch = -(-e // (NC * NS * K * GCH)) * GCH
    e_pad = NC * NS * K * nch
    pe = e_pad - e
    rows = jnp.concatenate([edge_index[0], jnp.zeros((pe,), edge_index.dtype)])
    cols = jnp.concatenate([edge_index[1], jnp.zeros((pe,), edge_index.dtype)])
    w = jnp.concatenate([edge_weight, jnp.zeros((pe,), edge_weight.dtype)])
    rows2d = rows.reshape(e_pad // K, K)
    cols2d = cols.reshape(e_pad // K, K)
    w8 = w.reshape(e_pad // 8, 8)
    w2d = jnp.concatenate([w8, w8], axis=1)

    spmm = _make_sc_spmm(n, d, e_pad)
    parts1 = spmm(features, cols2d, rows2d, w2d)

    bn = 1000 if n % 1000 == 0 else n
    l1f, inter = _tc_combine(parts1, features, bn)

    parts2 = spmm(inter, cols2d, rows2d, w2d)

    b = (b1 + b2).reshape(1, d)
    return _tc_final(l1f, parts2, W1, W2, b, bn)
